# baseline (speedup 1.0000x reference)
import functools

import jax
import jax.numpy as jnp
from jax import lax
from jax.experimental import pallas as pl
from jax.experimental.pallas import tpu as pltpu
from jax.experimental.pallas import tpu_sc as plsc

NC = 2
NS = 16
NW = NC * NS
L = 16

N = 1_000_000
VREGS = N // L
VPW = VREGS // NW
TAIL_VREGS = VREGS - VPW * NW
WORDS = VPW * 3 * L
TAIL_WORDS = TAIL_VREGS * 3 * L

CAP = 352
DRAIN_AT = 64
CHECK_EVERY = 16

INF = float("inf")


def _splat(x, dtype=jnp.float32):
    return jnp.full((L,), x, dtype=dtype)


def _merge_sorted(rv, ri, sv_desc, si_desc):
    m = sv_desc < rv
    nv = jnp.where(m, sv_desc, rv)
    ni = jnp.where(m, si_desc, ri)
    out = plsc.sort_key_val(nv, ni)
    return out[0], out[1]


def _topk_body(pc_ref, p1_ref, outv_ref, outi_ref, outx_ref, outy_ref,
               outz_ref, pts, p1v, candv, candi, stgv, stgi, stgx, stgy, stgz):
    wid = lax.axis_index("c") * NS + lax.axis_index("s")
    base_w = wid * WORDS

    pltpu.sync_copy(pc_ref.at[pl.ds(base_w, WORDS)], pts.at[pl.ds(0, WORDS)])
    pltpu.sync_copy(p1_ref, p1v)

    @pl.when(wid == 0)
    def _():
        pltpu.sync_copy(pc_ref.at[pl.ds(NW * WORDS, TAIL_WORDS)],
                        pts.at[pl.ds(WORDS, TAIL_WORDS)])

    def _fill(j, c):
        candv[pl.ds(j * L, L)] = _splat(INF)
        return c
    lax.fori_loop(0, CAP // L, _fill, 0)

    iota = lax.iota(jnp.int32, L)
    q = p1v[...]
    qx = jnp.full((L,), q[0], dtype=jnp.float32)
    qy = jnp.full((L,), q[1], dtype=jnp.float32)
    qz = jnp.full((L,), q[2], dtype=jnp.float32)

    def drain(rv, ri, off):
        off_s = jnp.max(off)
        nvregs = (off_s + L - 1) // L

        def body(j, c):
            rv, ri = c
            cv = candv[pl.ds(j * L, L)]
            ci = candi[pl.ds(j * L, L)]
            sv, si = plsc.sort_key_val(cv, ci, descending=True)
            rv, ri = _merge_sorted(rv, ri, sv, si)
            candv[pl.ds(j * L, L)] = _splat(INF)
            return rv, ri

        rv, ri = lax.fori_loop(0, nvregs, body, (rv, ri))
        t = jnp.full((L,), jnp.max(rv), dtype=jnp.float32)
        return rv, ri, t, jnp.zeros((L,), jnp.int32)

    def step(i, c):
        rv, ri, t, off, g, xi = c
        x = plsc.load_gather(pts, [xi])
        y = plsc.load_gather(pts, [xi + 1])
        z = plsc.load_gather(pts, [xi + 2])
        dx = x - qx
        dy = y - qy
        dz = z - qz
        d = dx * dx + dy * dy + dz * dz
        m = d < t
        m32 = m.astype(jnp.int32)
        pos = jnp.maximum(off + plsc.cumsum(m32) - 1, 0)
        plsc.store_scatter(candv, [pos], d, mask=m)
        plsc.store_scatter(candi, [pos], g, mask=m)
        off = off + plsc.all_reduce_population_count(m)

        def maybe_drain(c):
            rv, ri, t, off = c
            off_s = jnp.max(off)
            return lax.cond(off_s > DRAIN_AT,
                            lambda c2: drain(c2[0], c2[1], c2[3]),
                            lambda c2: c2, (rv, ri, t, off))

        rv, ri, t, off = lax.cond(i % CHECK_EVERY == CHECK_EVERY - 1,
                                  maybe_drain, lambda c2: c2,
                                  (rv, ri, t, off))
        return rv, ri, t, off, g + L, xi + 3 * L

    init = (_splat(INF), jnp.zeros((L,), jnp.int32), _splat(INF),
            jnp.zeros((L,), jnp.int32), wid * (VPW * L) + iota, iota * 3)
    carry = lax.fori_loop(0, VPW, step, init)

    def tail(c):
        rv, ri, t, off, g, xi = c
        g2 = _splat(NW * VPW * L, jnp.int32) + iota
        return lax.fori_loop(VPW, VPW + TAIL_VREGS, step,
                             (rv, ri, t, off, g2, xi))

    carry = lax.cond(wid == 0, tail, lambda c: c, carry)
    rv, ri, t, off, g, xi = carry
    rv, ri, t, off = drain(rv, ri, off)

    rel = jnp.where(ri >= NW * VPW * L, ri - (NW * VPW * L) + VPW * L,
                    ri - wid * (VPW * L))
    rel = jnp.clip(rel, 0, VPW * L + TAIL_VREGS * L - 1)
    relw = rel * 3
    px = plsc.load_gather(pts, [relw])
    py = plsc.load_gather(pts, [relw + 1])
    pz = plsc.load_gather(pts, [relw + 2])

    stgv[...] = rv
    stgi[...] = ri
    stgx[...] = px
    stgy[...] = py
    stgz[...] = pz
    pltpu.sync_copy(stgv, outv_ref.at[pl.ds(wid * L, L)])
    pltpu.sync_copy(stgi, outi_ref.at[pl.ds(wid * L, L)])
    pltpu.sync_copy(stgx, outx_ref.at[pl.ds(wid * L, L)])
    pltpu.sync_copy(stgy, outy_ref.at[pl.ds(wid * L, L)])
    pltpu.sync_copy(stgz, outz_ref.at[pl.ds(wid * L, L)])


def _merge_body(candv_ref, candi_ref, candx_ref, candy_ref, candz_ref,
                outp_ref, outi_ref, vbuf, ibuf, xbuf, ybuf, zbuf,
                rowsb, idxb):
    wid = lax.axis_index("c") * NS + lax.axis_index("s")

    @pl.when(wid == 0)
    def _():
        pltpu.sync_copy(candv_ref, vbuf)
        pltpu.sync_copy(candi_ref, ibuf)
        pltpu.sync_copy(candx_ref, xbuf)
        pltpu.sync_copy(candy_ref, ybuf)
        pltpu.sync_copy(candz_ref, zbuf)

        iota = lax.iota(jnp.int32, L)

        def body(j, c):
            rv, rp = c
            cv = jnp.flip(vbuf[pl.ds(j * L, L)])
            cp = jnp.flip(j * L + iota)
            return _merge_sorted(rv, rp, cv, cp)

        rv, rp = lax.fori_loop(0, NW, body,
                               (_splat(INF), jnp.zeros((L,), jnp.int32)))

        ri = plsc.load_gather(ibuf, [rp])
        px = plsc.load_gather(xbuf, [rp])
        py = plsc.load_gather(ybuf, [rp])
        pz = plsc.load_gather(zbuf, [rp])

        idxb[...] = ri
        pltpu.sync_copy(idxb, outi_ref)
        plsc.store_scatter(rowsb, [iota * 3], px)
        plsc.store_scatter(rowsb, [iota * 3 + 1], py)
        plsc.store_scatter(rowsb, [iota * 3 + 2], pz)
        pltpu.sync_copy(rowsb, outp_ref)


_mesh = plsc.VectorSubcoreMesh(core_axis_name="c", subcore_axis_name="s",
                               num_cores=NC, num_subcores=NS)

_params = pltpu.CompilerParams(needs_layout_passes=False)

_topk_call = pl.kernel(
    _topk_body,
    out_type=(jax.ShapeDtypeStruct((NW * L,), jnp.float32),
              jax.ShapeDtypeStruct((NW * L,), jnp.int32),
              jax.ShapeDtypeStruct((NW * L,), jnp.float32),
              jax.ShapeDtypeStruct((NW * L,), jnp.float32),
              jax.ShapeDtypeStruct((NW * L,), jnp.float32)),
    mesh=_mesh,
    compiler_params=_params,
    scratch_types=[
        pltpu.VMEM((WORDS + TAIL_WORDS,), jnp.float32),
        pltpu.VMEM((L,), jnp.float32),
        pltpu.VMEM((CAP,), jnp.float32),
        pltpu.VMEM((CAP,), jnp.int32),
        pltpu.VMEM((L,), jnp.float32),
        pltpu.VMEM((L,), jnp.int32),
        pltpu.VMEM((L,), jnp.float32),
        pltpu.VMEM((L,), jnp.float32),
        pltpu.VMEM((L,), jnp.float32),
    ],
)

_merge_call = pl.kernel(
    _merge_body,
    out_type=(jax.ShapeDtypeStruct((3 * L,), jnp.float32),
              jax.ShapeDtypeStruct((L,), jnp.int32)),
    mesh=_mesh,
    compiler_params=_params,
    scratch_types=[
        pltpu.VMEM((NW * L,), jnp.float32),
        pltpu.VMEM((NW * L,), jnp.int32),
        pltpu.VMEM((NW * L,), jnp.float32),
        pltpu.VMEM((NW * L,), jnp.float32),
        pltpu.VMEM((NW * L,), jnp.float32),
        pltpu.VMEM((3 * L,), jnp.float32),
        pltpu.VMEM((L,), jnp.int32),
    ],
)


def kernel(pcloud, P1, K):
    pc_flat = jnp.reshape(pcloud, (-1,))
    p1p = jnp.pad(jnp.asarray(P1, jnp.float32), (0, L - 3))
    cv, ci, cx, cy, cz = _topk_call(pc_flat, p1p)
    pts, idx = _merge_call(cv, ci, cx, cy, cz)
    idx = idx + (K - 16)
    return (jnp.reshape(pts, (L, 3)), jnp.reshape(idx, (1, L)))


# speedup vs baseline: 1.0126x; 1.0126x over previous
import functools

import jax
import jax.numpy as jnp
from jax import lax
from jax.experimental import pallas as pl
from jax.experimental.pallas import tpu as pltpu
from jax.experimental.pallas import tpu_sc as plsc

NC = 2
NS = 16
NW = NC * NS
L = 16

N = 1_000_000
VREGS = N // L
VPW = VREGS // NW
TAIL_VREGS = VREGS - VPW * NW
WORDS = VPW * 3 * L
TAIL_WORDS = TAIL_VREGS * 3 * L

U = 21
NBLK = VPW // U
CAP = 448
DRAIN_AT = 64

INF = float("inf")


def _splat(x, dtype=jnp.float32):
    return jnp.full((L,), x, dtype=dtype)


def _merge_sorted(rv, ri, sv_desc, si_desc):
    m = sv_desc < rv
    nv = jnp.where(m, sv_desc, rv)
    ni = jnp.where(m, si_desc, ri)
    out = plsc.sort_key_val(nv, ni)
    return out[0], out[1]


def _topk_body(pc_ref, p1_ref, outv_ref, outi_ref, outx_ref, outy_ref,
               outz_ref, pts, p1v, candv, candi, stgv, stgi, stgx, stgy, stgz):
    wid = lax.axis_index("c") * NS + lax.axis_index("s")
    base_w = wid * WORDS

    pltpu.sync_copy(pc_ref.at[pl.ds(base_w, WORDS)], pts.at[pl.ds(0, WORDS)])
    pltpu.sync_copy(p1_ref, p1v)

    @pl.when(wid == 0)
    def _():
        pltpu.sync_copy(pc_ref.at[pl.ds(NW * WORDS, TAIL_WORDS)],
                        pts.at[pl.ds(WORDS, TAIL_WORDS)])

    def _fill(j, c):
        candv[pl.ds(j * L, L)] = _splat(INF)
        return c
    lax.fori_loop(0, CAP // L, _fill, 0)

    iota = lax.iota(jnp.int32, L)
    q = p1v[...]
    qx = jnp.full((L,), q[0], dtype=jnp.float32)
    qy = jnp.full((L,), q[1], dtype=jnp.float32)
    qz = jnp.full((L,), q[2], dtype=jnp.float32)

    def drain(rv, ri, off):
        nvregs = (off + L - 1) // L

        def body(j, c):
            rv, ri = c
            cv = candv[pl.ds(j * L, L)]
            ci = candi[pl.ds(j * L, L)]
            sv, si = plsc.sort_key_val(cv, ci, descending=True)
            rv, ri = _merge_sorted(rv, ri, sv, si)
            candv[pl.ds(j * L, L)] = _splat(INF)
            return rv, ri

        rv, ri = lax.fori_loop(0, nvregs, body, (rv, ri))
        t = jnp.full((L,), jnp.max(rv), dtype=jnp.float32)
        return rv, ri, t, jnp.int32(0)

    def step(t, off, g, xi):
        x = plsc.load_gather(pts, [xi])
        y = plsc.load_gather(pts, [xi + 1])
        z = plsc.load_gather(pts, [xi + 2])
        dx = x - qx
        dy = y - qy
        dz = z - qz
        d = dx * dx + dy * dy + dz * dz
        m = d < t
        plsc.store_compressed(candv.at[pl.ds(off, L)], d, mask=m)
        plsc.store_compressed(candi.at[pl.ds(off, L)], g, mask=m)
        return off + plsc.all_reduce_population_count(m)[0]

    def block(b, c):
        rv, ri, t, off, g0, xi0 = c
        for j in range(U):
            off = step(t, off, g0 + j * L, xi0 + j * (3 * L))
        rv, ri, t, off = lax.cond(
            off > DRAIN_AT,
            lambda c2: drain(c2[0], c2[1], c2[3]),
            lambda c2: c2, (rv, ri, t, off))
        return rv, ri, t, off, g0 + U * L, xi0 + U * 3 * L

    init = (_splat(INF), jnp.zeros((L,), jnp.int32), _splat(INF),
            jnp.int32(0), wid * (VPW * L) + iota, iota * 3)
    carry = lax.fori_loop(0, NBLK, block, init)

    def tail(c):
        rv, ri, t, off, g0, xi0 = c
        g2 = _splat(NW * VPW * L, jnp.int32) + iota
        for j in range(TAIL_VREGS):
            off = step(t, off, g2 + j * L, xi0 + j * (3 * L))
        return rv, ri, t, off, g0, xi0

    carry = lax.cond(wid == 0, tail, lambda c: c, carry)
    rv, ri, t, off, g0, xi0 = carry
    rv, ri, t, off = drain(rv, ri, off)

    rel = jnp.where(ri >= NW * VPW * L, ri - (NW * VPW * L) + VPW * L,
                    ri - wid * (VPW * L))
    rel = jnp.clip(rel, 0, VPW * L + TAIL_VREGS * L - 1)
    relw = rel * 3
    px = plsc.load_gather(pts, [relw])
    py = plsc.load_gather(pts, [relw + 1])
    pz = plsc.load_gather(pts, [relw + 2])

    stgv[...] = rv
    stgi[...] = ri
    stgx[...] = px
    stgy[...] = py
    stgz[...] = pz
    pltpu.sync_copy(stgv, outv_ref.at[pl.ds(wid * L, L)])
    pltpu.sync_copy(stgi, outi_ref.at[pl.ds(wid * L, L)])
    pltpu.sync_copy(stgx, outx_ref.at[pl.ds(wid * L, L)])
    pltpu.sync_copy(stgy, outy_ref.at[pl.ds(wid * L, L)])
    pltpu.sync_copy(stgz, outz_ref.at[pl.ds(wid * L, L)])


def _merge_body(candv_ref, candi_ref, candx_ref, candy_ref, candz_ref,
                outp_ref, outi_ref, vbuf, ibuf, xbuf, ybuf, zbuf,
                rowsb, idxb):
    wid = lax.axis_index("c") * NS + lax.axis_index("s")

    @pl.when(wid == 0)
    def _():
        pltpu.sync_copy(candv_ref, vbuf)
        pltpu.sync_copy(candi_ref, ibuf)
        pltpu.sync_copy(candx_ref, xbuf)
        pltpu.sync_copy(candy_ref, ybuf)
        pltpu.sync_copy(candz_ref, zbuf)

        iota = lax.iota(jnp.int32, L)

        rv, rp = _splat(INF), jnp.zeros((L,), jnp.int32)
        for j in range(NW):
            cv = jnp.flip(vbuf[pl.ds(j * L, L)])
            cp = jnp.flip(j * L + iota)
            rv, rp = _merge_sorted(rv, rp, cv, cp)

        ri = plsc.load_gather(ibuf, [rp])
        px = plsc.load_gather(xbuf, [rp])
        py = plsc.load_gather(ybuf, [rp])
        pz = plsc.load_gather(zbuf, [rp])

        idxb[...] = ri
        pltpu.sync_copy(idxb, outi_ref)
        plsc.store_scatter(rowsb, [iota * 3], px)
        plsc.store_scatter(rowsb, [iota * 3 + 1], py)
        plsc.store_scatter(rowsb, [iota * 3 + 2], pz)
        pltpu.sync_copy(rowsb, outp_ref)


_mesh = plsc.VectorSubcoreMesh(core_axis_name="c", subcore_axis_name="s",
                               num_cores=NC, num_subcores=NS)

_params = pltpu.CompilerParams(needs_layout_passes=False)

_topk_call = pl.kernel(
    _topk_body,
    out_type=(jax.ShapeDtypeStruct((NW * L,), jnp.float32),
              jax.ShapeDtypeStruct((NW * L,), jnp.int32),
              jax.ShapeDtypeStruct((NW * L,), jnp.float32),
              jax.ShapeDtypeStruct((NW * L,), jnp.float32),
              jax.ShapeDtypeStruct((NW * L,), jnp.float32)),
    mesh=_mesh,
    compiler_params=_params,
    scratch_types=[
        pltpu.VMEM((WORDS + TAIL_WORDS,), jnp.float32),
        pltpu.VMEM((L,), jnp.float32),
        pltpu.VMEM((CAP,), jnp.float32),
        pltpu.VMEM((CAP,), jnp.int32),
        pltpu.VMEM((L,), jnp.float32),
        pltpu.VMEM((L,), jnp.int32),
        pltpu.VMEM((L,), jnp.float32),
        pltpu.VMEM((L,), jnp.float32),
        pltpu.VMEM((L,), jnp.float32),
    ],
)

_merge_call = pl.kernel(
    _merge_body,
    out_type=(jax.ShapeDtypeStruct((3 * L,), jnp.float32),
              jax.ShapeDtypeStruct((L,), jnp.int32)),
    mesh=_mesh,
    compiler_params=_params,
    scratch_types=[
        pltpu.VMEM((NW * L,), jnp.float32),
        pltpu.VMEM((NW * L,), jnp.int32),
        pltpu.VMEM((NW * L,), jnp.float32),
        pltpu.VMEM((NW * L,), jnp.float32),
        pltpu.VMEM((NW * L,), jnp.float32),
        pltpu.VMEM((3 * L,), jnp.float32),
        pltpu.VMEM((L,), jnp.int32),
    ],
)


def kernel(pcloud, P1, K):
    pc_flat = jnp.reshape(pcloud, (-1,))
    p1p = jnp.pad(jnp.asarray(P1, jnp.float32), (0, L - 3))
    cv, ci, cx, cy, cz = _topk_call(pc_flat, p1p)
    pts, idx = _merge_call(cv, ci, cx, cy, cz)
    idx = idx + (K - 16)
    return (jnp.reshape(pts, (L, 3)), jnp.reshape(idx, (1, L)))


# speedup vs baseline: 11.1546x; 11.0159x over previous
import jax
import jax.numpy as jnp
from jax import lax
from jax.experimental import pallas as pl
from jax.experimental.pallas import tpu as pltpu
from jax.experimental.pallas import tpu_sc as plsc

NC = 2
NS = 16
NW = NC * NS
L = 16

N = 1_000_000
VREGS = N // L
VPW = VREGS // NW
TAIL_VREGS = VREGS - VPW * NW
PW = VPW * L
TW = TAIL_VREGS * L

U = 21
NBLK = VPW // U
CAP = 448
DRAIN_AT = 64

INF = float("inf")


def _splat(x, dtype=jnp.float32):
    return jnp.full((L,), x, dtype=dtype)


def _merge_sorted(rv, ri, sv_desc, si_desc):
    m = sv_desc < rv
    nv = jnp.where(m, sv_desc, rv)
    ni = jnp.where(m, si_desc, ri)
    out = plsc.sort_key_val(nv, ni)
    return out[0], out[1]


def _topk_body(px_ref, py_ref, pz_ref, p1_ref,
               outv_ref, outi_ref, outx_ref, outy_ref, outz_ref,
               xb, yb, zb, p1v, candv, candi,
               stgv, stgi, stgx, stgy, stgz):
    wid = lax.axis_index("c") * NS + lax.axis_index("s")
    base = wid * PW

    pltpu.sync_copy(px_ref.at[pl.ds(base, PW)], xb.at[pl.ds(0, PW)])
    pltpu.sync_copy(py_ref.at[pl.ds(base, PW)], yb.at[pl.ds(0, PW)])
    pltpu.sync_copy(pz_ref.at[pl.ds(base, PW)], zb.at[pl.ds(0, PW)])
    pltpu.sync_copy(p1_ref, p1v)

    @pl.when(wid == 0)
    def _():
        pltpu.sync_copy(px_ref.at[pl.ds(NW * PW, TW)], xb.at[pl.ds(PW, TW)])
        pltpu.sync_copy(py_ref.at[pl.ds(NW * PW, TW)], yb.at[pl.ds(PW, TW)])
        pltpu.sync_copy(pz_ref.at[pl.ds(NW * PW, TW)], zb.at[pl.ds(PW, TW)])

    def _fill(j, c):
        candv[pl.ds(j * L, L)] = _splat(INF)
        return c
    lax.fori_loop(0, CAP // L, _fill, 0)

    iota = lax.iota(jnp.int32, L)
    q = p1v[...]
    qx = jnp.full((L,), q[0], dtype=jnp.float32)
    qy = jnp.full((L,), q[1], dtype=jnp.float32)
    qz = jnp.full((L,), q[2], dtype=jnp.float32)

    def drain(rv, ri, off):
        nvregs = (off + L - 1) // L

        def body(j, c):
            rv, ri = c
            cv = candv[pl.ds(j * L, L)]
            ci = candi[pl.ds(j * L, L)]
            sv, si = plsc.sort_key_val(cv, ci, descending=True)
            rv, ri = _merge_sorted(rv, ri, sv, si)
            candv[pl.ds(j * L, L)] = _splat(INF)
            return rv, ri

        rv, ri = lax.fori_loop(0, nvregs, body, (rv, ri))
        t = jnp.full((L,), jnp.max(rv), dtype=jnp.float32)
        return rv, ri, t, jnp.int32(0)

    def step(t, off, g, w):
        x = xb[pl.ds(w, L)]
        y = yb[pl.ds(w, L)]
        z = zb[pl.ds(w, L)]
        dx = x - qx
        dy = y - qy
        dz = z - qz
        d = dx * dx + dy * dy + dz * dz
        m = d < t
        plsc.store_compressed(candv.at[pl.ds(off, L)], d, mask=m)
        plsc.store_compressed(candi.at[pl.ds(off, L)], g, mask=m)
        return off + plsc.all_reduce_population_count(m)[0]

    def block(b, c):
        rv, ri, t, off, g0 = c
        w0 = b * (U * L)
        for j in range(U):
            off = step(t, off, g0 + j * L, w0 + j * L)
        rv, ri, t, off = lax.cond(
            off > DRAIN_AT,
            lambda c2: drain(c2[0], c2[1], c2[3]),
            lambda c2: c2, (rv, ri, t, off))
        return rv, ri, t, off, g0 + U * L

    init = (_splat(INF), jnp.zeros((L,), jnp.int32), _splat(INF),
            jnp.int32(0), base + iota)
    carry = lax.fori_loop(0, NBLK, block, init)

    def tail(c):
        rv, ri, t, off, g0 = c
        g2 = _splat(NW * PW, jnp.int32) + iota
        for j in range(TAIL_VREGS):
            off = step(t, off, g2 + j * L, PW + j * L)
        return rv, ri, t, off, g0

    carry = lax.cond(wid == 0, tail, lambda c: c, carry)
    rv, ri, t, off, g0 = carry
    rv, ri, t, off = drain(rv, ri, off)

    rel = jnp.where(ri >= NW * PW, ri - (NW * PW) + PW, ri - base)
    rel = jnp.clip(rel, 0, PW + TW - 1)
    px = plsc.load_gather(xb, [rel])
    py = plsc.load_gather(yb, [rel])
    pz = plsc.load_gather(zb, [rel])

    stgv[...] = rv
    stgi[...] = ri
    stgx[...] = px
    stgy[...] = py
    stgz[...] = pz
    pltpu.sync_copy(stgv, outv_ref.at[pl.ds(wid * L, L)])
    pltpu.sync_copy(stgi, outi_ref.at[pl.ds(wid * L, L)])
    pltpu.sync_copy(stgx, outx_ref.at[pl.ds(wid * L, L)])
    pltpu.sync_copy(stgy, outy_ref.at[pl.ds(wid * L, L)])
    pltpu.sync_copy(stgz, outz_ref.at[pl.ds(wid * L, L)])


def _merge_body(candv_ref, candi_ref, candx_ref, candy_ref, candz_ref,
                outp_ref, outi_ref, vbuf, ibuf, xbuf, ybuf, zbuf,
                rowsb, idxb):
    wid = lax.axis_index("c") * NS + lax.axis_index("s")

    @pl.when(wid == 0)
    def _():
        pltpu.sync_copy(candv_ref, vbuf)
        pltpu.sync_copy(candi_ref, ibuf)
        pltpu.sync_copy(candx_ref, xbuf)
        pltpu.sync_copy(candy_ref, ybuf)
        pltpu.sync_copy(candz_ref, zbuf)

        iota = lax.iota(jnp.int32, L)

        rv, rp = _splat(INF), jnp.zeros((L,), jnp.int32)
        for j in range(NW):
            cv = jnp.flip(vbuf[pl.ds(j * L, L)])
            cp = jnp.flip(j * L + iota)
            rv, rp = _merge_sorted(rv, rp, cv, cp)

        ri = plsc.load_gather(ibuf, [rp])
        px = plsc.load_gather(xbuf, [rp])
        py = plsc.load_gather(ybuf, [rp])
        pz = plsc.load_gather(zbuf, [rp])

        idxb[...] = ri
        pltpu.sync_copy(idxb, outi_ref)
        plsc.store_scatter(rowsb, [iota * 3], px)
        plsc.store_scatter(rowsb, [iota * 3 + 1], py)
        plsc.store_scatter(rowsb, [iota * 3 + 2], pz)
        pltpu.sync_copy(rowsb, outp_ref)


_mesh = plsc.VectorSubcoreMesh(core_axis_name="c", subcore_axis_name="s",
                               num_cores=NC, num_subcores=NS)

_params = pltpu.CompilerParams(needs_layout_passes=False)

_topk_call = pl.kernel(
    _topk_body,
    out_type=(jax.ShapeDtypeStruct((NW * L,), jnp.float32),
              jax.ShapeDtypeStruct((NW * L,), jnp.int32),
              jax.ShapeDtypeStruct((NW * L,), jnp.float32),
              jax.ShapeDtypeStruct((NW * L,), jnp.float32),
              jax.ShapeDtypeStruct((NW * L,), jnp.float32)),
    mesh=_mesh,
    compiler_params=_params,
    scratch_types=[
        pltpu.VMEM((PW + TW,), jnp.float32),
        pltpu.VMEM((PW + TW,), jnp.float32),
        pltpu.VMEM((PW + TW,), jnp.float32),
        pltpu.VMEM((L,), jnp.float32),
        pltpu.VMEM((CAP,), jnp.float32),
        pltpu.VMEM((CAP,), jnp.int32),
        pltpu.VMEM((L,), jnp.float32),
        pltpu.VMEM((L,), jnp.int32),
        pltpu.VMEM((L,), jnp.float32),
        pltpu.VMEM((L,), jnp.float32),
        pltpu.VMEM((L,), jnp.float32),
    ],
)

_merge_call = pl.kernel(
    _merge_body,
    out_type=(jax.ShapeDtypeStruct((3 * L,), jnp.float32),
              jax.ShapeDtypeStruct((L,), jnp.int32)),
    mesh=_mesh,
    compiler_params=_params,
    scratch_types=[
        pltpu.VMEM((NW * L,), jnp.float32),
        pltpu.VMEM((NW * L,), jnp.int32),
        pltpu.VMEM((NW * L,), jnp.float32),
        pltpu.VMEM((NW * L,), jnp.float32),
        pltpu.VMEM((NW * L,), jnp.float32),
        pltpu.VMEM((3 * L,), jnp.float32),
        pltpu.VMEM((L,), jnp.int32),
    ],
)


def kernel(pcloud, P1, K):
    pc = jnp.reshape(pcloud, (N, 3))
    px = lax.squeeze(lax.slice(pc, (0, 0), (N, 1)), (1,))
    py = lax.squeeze(lax.slice(pc, (0, 1), (N, 2)), (1,))
    pz = lax.squeeze(lax.slice(pc, (0, 2), (N, 3)), (1,))
    p1p = jnp.pad(jnp.asarray(P1, jnp.float32), (0, L - 3))
    cv, ci, cx, cy, cz = _topk_call(px, py, pz, p1p)
    pts, idx = _merge_call(cv, ci, cx, cy, cz)
    idx = idx + (K - 16)
    return (jnp.reshape(pts, (L, 3)), jnp.reshape(idx, (1, L)))


# speedup vs baseline: 30.6526x; 2.7480x over previous
import jax
import jax.numpy as jnp
from jax import lax
from jax.experimental import pallas as pl
from jax.experimental.pallas import tpu as pltpu
from jax.experimental.pallas import tpu_sc as plsc

NC = 2
NS = 16
NW = NC * NS
L = 16

N = 1_000_000
VREGS = N // L
VPW = VREGS // NW
TAIL_VREGS = VREGS - VPW * NW
PW = VPW * L
TW = TAIL_VREGS * L

U = 21
NBLK = VPW // U
CAP = 448
DRAIN_AT = 64

INF = float("inf")


def _splat(x, dtype=jnp.float32):
    return jnp.full((L,), x, dtype=dtype)


def _merge_sorted(rv, ri, sv_desc, si_desc):
    m = sv_desc < rv
    nv = jnp.where(m, sv_desc, rv)
    ni = jnp.where(m, si_desc, ri)
    out = plsc.sort_key_val(nv, ni)
    return out[0], out[1]


def _topk_body(px_ref, py_ref, pz_ref, p1_ref,
               outv_ref, outi_ref, outx_ref, outy_ref, outz_ref,
               xb, yb, zb, p1v, candv, candi,
               stgv, stgi, stgx, stgy, stgz, dsem):
    wid = lax.axis_index("c") * NS + lax.axis_index("s")
    base = wid * PW

    cpx = pltpu.async_copy(px_ref.at[pl.ds(base, PW)], xb.at[pl.ds(0, PW)],
                           dsem)
    cpy = pltpu.async_copy(py_ref.at[pl.ds(base, PW)], yb.at[pl.ds(0, PW)],
                           dsem)
    cpz = pltpu.async_copy(pz_ref.at[pl.ds(base, PW)], zb.at[pl.ds(0, PW)],
                           dsem)
    pltpu.sync_copy(p1_ref, p1v)

    @pl.when(wid == 0)
    def _():
        pltpu.sync_copy(px_ref.at[pl.ds(NW * PW, TW)], xb.at[pl.ds(PW, TW)])
        pltpu.sync_copy(py_ref.at[pl.ds(NW * PW, TW)], yb.at[pl.ds(PW, TW)])
        pltpu.sync_copy(pz_ref.at[pl.ds(NW * PW, TW)], zb.at[pl.ds(PW, TW)])

    cpx.wait()
    cpy.wait()
    cpz.wait()

    def _fill(j, c):
        candv[pl.ds(j * L, L)] = _splat(INF)
        return c
    lax.fori_loop(0, CAP // L, _fill, 0)

    iota = lax.iota(jnp.int32, L)
    q = p1v[...]
    qx = jnp.full((L,), q[0], dtype=jnp.float32)
    qy = jnp.full((L,), q[1], dtype=jnp.float32)
    qz = jnp.full((L,), q[2], dtype=jnp.float32)

    def drain(rv, ri, off):
        nvregs = (off + L - 1) // L

        def body(j, c):
            rv, ri = c
            cv = candv[pl.ds(j * L, L)]
            ci = candi[pl.ds(j * L, L)]
            sv, si = plsc.sort_key_val(cv, ci, descending=True)
            rv, ri = _merge_sorted(rv, ri, sv, si)
            candv[pl.ds(j * L, L)] = _splat(INF)
            return rv, ri

        rv, ri = lax.fori_loop(0, nvregs, body, (rv, ri))
        t = jnp.full((L,), jnp.max(rv), dtype=jnp.float32)
        return rv, ri, t, jnp.int32(0)

    def step(t, off, g, w):
        x = xb[pl.ds(w, L)]
        y = yb[pl.ds(w, L)]
        z = zb[pl.ds(w, L)]
        dx = x - qx
        dy = y - qy
        dz = z - qz
        d = dx * dx + dy * dy + dz * dz
        m = d < t
        plsc.store_compressed(candv.at[pl.ds(off, L)], d, mask=m)
        plsc.store_compressed(candi.at[pl.ds(off, L)], g, mask=m)
        return off + plsc.all_reduce_population_count(m)[0]

    def block(b, c):
        rv, ri, t, off, g0 = c
        w0 = b * (U * L)
        for j in range(U):
            off = step(t, off, g0 + j * L, w0 + j * L)
        rv, ri, t, off = lax.cond(
            off > DRAIN_AT,
            lambda c2: drain(c2[0], c2[1], c2[3]),
            lambda c2: c2, (rv, ri, t, off))
        return rv, ri, t, off, g0 + U * L

    init = (_splat(INF), jnp.zeros((L,), jnp.int32), _splat(INF),
            jnp.int32(0), base + iota)
    carry = lax.fori_loop(0, NBLK, block, init)

    def tail(c):
        rv, ri, t, off, g0 = c
        g2 = _splat(NW * PW, jnp.int32) + iota
        for j in range(TAIL_VREGS):
            off = step(t, off, g2 + j * L, PW + j * L)
        return rv, ri, t, off, g0

    carry = lax.cond(wid == 0, tail, lambda c: c, carry)
    rv, ri, t, off, g0 = carry
    rv, ri, t, off = drain(rv, ri, off)

    rel = jnp.where(ri >= NW * PW, ri - (NW * PW) + PW, ri - base)
    rel = jnp.clip(rel, 0, PW + TW - 1)
    px = plsc.load_gather(xb, [rel])
    py = plsc.load_gather(yb, [rel])
    pz = plsc.load_gather(zb, [rel])

    stgv[...] = rv
    stgi[...] = ri
    stgx[...] = px
    stgy[...] = py
    stgz[...] = pz
    pltpu.sync_copy(stgv, outv_ref.at[pl.ds(wid * L, L)])
    pltpu.sync_copy(stgi, outi_ref.at[pl.ds(wid * L, L)])
    pltpu.sync_copy(stgx, outx_ref.at[pl.ds(wid * L, L)])
    pltpu.sync_copy(stgy, outy_ref.at[pl.ds(wid * L, L)])
    pltpu.sync_copy(stgz, outz_ref.at[pl.ds(wid * L, L)])


def _merge_body(candv_ref, candi_ref, candx_ref, candy_ref, candz_ref,
                outp_ref, outi_ref, vbuf, ibuf, xbuf, ybuf, zbuf,
                rowsb, idxb):
    wid = lax.axis_index("c") * NS + lax.axis_index("s")

    @pl.when(wid == 0)
    def _():
        pltpu.sync_copy(candv_ref, vbuf)
        pltpu.sync_copy(candi_ref, ibuf)
        pltpu.sync_copy(candx_ref, xbuf)
        pltpu.sync_copy(candy_ref, ybuf)
        pltpu.sync_copy(candz_ref, zbuf)

        iota = lax.iota(jnp.int32, L)

        rv, rp = _splat(INF), jnp.zeros((L,), jnp.int32)
        for j in range(NW):
            cv = jnp.flip(vbuf[pl.ds(j * L, L)])
            cp = jnp.flip(j * L + iota)
            rv, rp = _merge_sorted(rv, rp, cv, cp)

        ri = plsc.load_gather(ibuf, [rp])
        px = plsc.load_gather(xbuf, [rp])
        py = plsc.load_gather(ybuf, [rp])
        pz = plsc.load_gather(zbuf, [rp])

        idxb[...] = ri
        pltpu.sync_copy(idxb, outi_ref)
        plsc.store_scatter(rowsb, [iota * 3], px)
        plsc.store_scatter(rowsb, [iota * 3 + 1], py)
        plsc.store_scatter(rowsb, [iota * 3 + 2], pz)
        pltpu.sync_copy(rowsb, outp_ref)


_mesh = plsc.VectorSubcoreMesh(core_axis_name="c", subcore_axis_name="s",
                               num_cores=NC, num_subcores=NS)

_params = pltpu.CompilerParams(needs_layout_passes=False)

_topk_call = pl.kernel(
    _topk_body,
    out_type=(jax.ShapeDtypeStruct((NW * L,), jnp.float32),
              jax.ShapeDtypeStruct((NW * L,), jnp.int32),
              jax.ShapeDtypeStruct((NW * L,), jnp.float32),
              jax.ShapeDtypeStruct((NW * L,), jnp.float32),
              jax.ShapeDtypeStruct((NW * L,), jnp.float32)),
    mesh=_mesh,
    compiler_params=_params,
    scratch_types=[
        pltpu.VMEM((PW + TW,), jnp.float32),
        pltpu.VMEM((PW + TW,), jnp.float32),
        pltpu.VMEM((PW + TW,), jnp.float32),
        pltpu.VMEM((L,), jnp.float32),
        pltpu.VMEM((CAP,), jnp.float32),
        pltpu.VMEM((CAP,), jnp.int32),
        pltpu.VMEM((L,), jnp.float32),
        pltpu.VMEM((L,), jnp.int32),
        pltpu.VMEM((L,), jnp.float32),
        pltpu.VMEM((L,), jnp.float32),
        pltpu.VMEM((L,), jnp.float32),
        pltpu.SemaphoreType.DMA,
    ],
)

_merge_call = pl.kernel(
    _merge_body,
    out_type=(jax.ShapeDtypeStruct((3 * L,), jnp.float32),
              jax.ShapeDtypeStruct((L,), jnp.int32)),
    mesh=_mesh,
    compiler_params=_params,
    scratch_types=[
        pltpu.VMEM((NW * L,), jnp.float32),
        pltpu.VMEM((NW * L,), jnp.int32),
        pltpu.VMEM((NW * L,), jnp.float32),
        pltpu.VMEM((NW * L,), jnp.float32),
        pltpu.VMEM((NW * L,), jnp.float32),
        pltpu.VMEM((3 * L,), jnp.float32),
        pltpu.VMEM((L,), jnp.int32),
    ],
)


def kernel(pcloud, P1, K):
    px = jnp.reshape(lax.slice(pcloud, (0, 0, 0), (1, N, 1)), (N,))
    py = jnp.reshape(lax.slice(pcloud, (0, 0, 1), (1, N, 2)), (N,))
    pz = jnp.reshape(lax.slice(pcloud, (0, 0, 2), (1, N, 3)), (N,))
    p1p = jnp.pad(jnp.asarray(P1, jnp.float32), (0, L - 3))
    cv, ci, cx, cy, cz = _topk_call(px, py, pz, p1p)
    pts, idx = _merge_call(cv, ci, cx, cy, cz)
    idx = idx + (K - 16)
    return (jnp.reshape(pts, (L, 3)), jnp.reshape(idx, (1, L)))
